# Initial kernel scaffold; baseline (speedup 1.0000x reference)
import functools

import jax
import jax.numpy as jnp
from jax import lax
from jax.experimental import pallas as pl
from jax.experimental.pallas import tpu as pltpu
from jax.experimental.pallas import tpu_sc as plsc

_B = 16
_S = 512
_C = 2
_T = 4096
_L = 16

_mesh = plsc.VectorSubcoreMesh(core_axis_name="c", subcore_axis_name="s")


@functools.partial(
    pl.kernel,
    out_type=jax.ShapeDtypeStruct((_C * _B, _T), jnp.float32),
    mesh=_mesh,
    scratch_types=[
        pltpu.VMEM((_S + _L,), jnp.int32),
        pltpu.VMEM((_S * _C,), jnp.float32),
        pltpu.VMEM((_T,), jnp.int32),
        pltpu.VMEM((_T,), jnp.float32),
    ],
)
def _decode(vals_hbm, starts_hbm, out_hbm, starts_v, vals_v, m_v, out_v):
    comp = lax.axis_index("c")
    b = lax.axis_index("s")

    pltpu.sync_copy(starts_hbm.at[b], starts_v.at[pl.ds(0, _S)])
    pltpu.sync_copy(vals_hbm.at[b], vals_v)

    iota = lax.iota(jnp.int32, _L)
    starts_v[pl.ds(_S, _L)] = jnp.full((_L,), _T, jnp.int32)

    def init_body(i, carry):
        m_v[pl.ds(i * _L, _L)] = jnp.full((_L,), -1, jnp.int32)
        return carry

    lax.fori_loop(0, _T // _L, init_body, 0)

    def scat_body(g, carry):
        base = g * _L
        st = starts_v[pl.ds(base, _L)]
        nxt = plsc.load_gather(starts_v, [iota + (base + 1)])
        vis = nxt > st
        plsc.store_scatter(m_v, [st], iota + base, mask=vis)
        return carry

    lax.fori_loop(0, _S // _L, scat_body, 0)

    def scan_body(i, carry):
        mv = m_v[pl.ds(i * _L, _L)]
        sc = jnp.maximum(plsc.cummax(mv), carry)
        valid = sc >= 0
        idx = jnp.maximum(sc, 0) * _C + comp
        v = jnp.where(valid, plsc.load_gather(vals_v, [idx]), 0.0)
        out_v[pl.ds(i * _L, _L)] = v
        return jnp.max(sc)

    lax.fori_loop(0, _T // _L, scan_body, jnp.int32(-1))

    pltpu.sync_copy(out_v, out_hbm.at[comp * _B + b])


def kernel(segment_values, segment_starts):
    vals_flat = segment_values.reshape(_B, _S * _C)
    out = _decode(vals_flat, segment_starts)
    return out.reshape(_C, _B, _T)


# speedup vs baseline: 281.2305x; 281.2305x over previous
import functools

import jax
import jax.numpy as jnp
from jax import lax
from jax.experimental import pallas as pl
from jax.experimental.pallas import tpu as pltpu
from jax.experimental.pallas import tpu_sc as plsc

_B = 16
_S = 512
_C = 2
_T = 4096
_L = 16

_mesh = plsc.VectorSubcoreMesh(core_axis_name="c", subcore_axis_name="s")


@functools.partial(
    pl.kernel,
    out_type=jax.ShapeDtypeStruct((_C * _B, _T), jnp.float32),
    mesh=_mesh,
    compiler_params=pltpu.CompilerParams(needs_layout_passes=False),
    scratch_types=[
        pltpu.VMEM((_S + 128,), jnp.int32),
        pltpu.VMEM((_S * _C,), jnp.float32),
        pltpu.VMEM((_T,), jnp.int32),
        pltpu.VMEM((_T,), jnp.float32),
    ],
)
def _decode(vals_hbm, starts_hbm, out_hbm, starts_v, vals_v, m_v, out_v):
    comp = lax.axis_index("c")
    b = lax.axis_index("s")

    pltpu.sync_copy(starts_hbm.at[b], starts_v.at[pl.ds(0, _S)])
    pltpu.sync_copy(vals_hbm.at[b], vals_v)

    iota = lax.iota(jnp.int32, _L)
    for p in range(128 // _L):
        starts_v[pl.ds(_S + p * _L, _L)] = jnp.full((_L,), _T, jnp.int32)

    def init_body(i, carry):
        m_v[pl.ds(i * _L, _L)] = jnp.full((_L,), -1, jnp.int32)
        return carry

    lax.fori_loop(0, _T // _L, init_body, 0)

    def scat_body(g, carry):
        base = g * _L
        st = starts_v[pl.ds(base, _L)]
        nxt = plsc.load_gather(starts_v, [iota + (base + 1)])
        vis = nxt > st
        plsc.store_scatter(m_v, [st], iota + base, mask=vis)
        return carry

    lax.fori_loop(0, _S // _L, scat_body, 0)

    def scan_body(i, carry):
        mv = m_v[pl.ds(i * _L, _L)]
        sc = jnp.maximum(plsc.cummax(mv), carry)
        valid = sc >= 0
        idx = jnp.maximum(sc, 0) * _C + comp
        v = jnp.where(valid, plsc.load_gather(vals_v, [idx]), 0.0)
        out_v[pl.ds(i * _L, _L)] = v
        return jnp.max(sc)

    lax.fori_loop(0, _T // _L, scan_body, jnp.int32(-1))

    pltpu.sync_copy(out_v, out_hbm.at[comp * _B + b])


def kernel(segment_values, segment_starts):
    vals_flat = segment_values.reshape(_B, _S * _C)
    out = _decode(vals_flat, segment_starts)
    return out.reshape(_C, _B, _T)


# speedup vs baseline: 286.6427x; 1.0192x over previous
import functools

import jax
import jax.numpy as jnp
from jax import lax
from jax.experimental import pallas as pl
from jax.experimental.pallas import tpu as pltpu
from jax.experimental.pallas import tpu_sc as plsc

_B = 16
_S = 512
_C = 2
_T = 4096
_L = 16

_mesh = plsc.VectorSubcoreMesh(core_axis_name="c", subcore_axis_name="s")


@functools.partial(
    pl.kernel,
    out_type=jax.ShapeDtypeStruct((_C * _B, _T), jnp.float32),
    mesh=_mesh,
    compiler_params=pltpu.CompilerParams(needs_layout_passes=False),
    scratch_types=[
        pltpu.VMEM((_S + 128,), jnp.int32),
        pltpu.VMEM((_S * _C,), jnp.float32),
        pltpu.VMEM((_T,), jnp.int32),
        pltpu.VMEM((_T,), jnp.float32),
    ],
)
def _decode(vals_hbm, starts_hbm, out_hbm, starts_v, vals_v, m_v, out_v):
    comp = lax.axis_index("c")
    b = lax.axis_index("s")

    pltpu.sync_copy(starts_hbm.at[b], starts_v.at[pl.ds(0, _S)])
    pltpu.sync_copy(vals_hbm.at[b], vals_v)

    iota = lax.iota(jnp.int32, _L)
    for p in range(128 // _L):
        starts_v[pl.ds(_S + p * _L, _L)] = jnp.full((_L,), _T, jnp.int32)

    def init_body(i, carry):
        m_v[pl.ds(i * _L, _L)] = jnp.full((_L,), -1, jnp.int32)
        return carry

    lax.fori_loop(0, _T // _L, init_body, 0, unroll=8)

    def scat_body(g, carry):
        base = g * _L
        st = starts_v[pl.ds(base, _L)]
        nxt = plsc.load_gather(starts_v, [iota + (base + 1)])
        vis = nxt > st
        plsc.store_scatter(m_v, [st], iota + base, mask=vis)
        return carry

    lax.fori_loop(0, _S // _L, scat_body, 0, unroll=4)

    def scan_body(i, carry):
        mv = m_v[pl.ds(i * _L, _L)]
        sc = jnp.maximum(plsc.cummax(mv), carry)
        valid = sc >= 0
        idx = jnp.maximum(sc, 0) * _C + comp
        v = jnp.where(valid, plsc.load_gather(vals_v, [idx]), 0.0)
        out_v[pl.ds(i * _L, _L)] = v
        return jnp.max(sc)

    lax.fori_loop(0, _T // _L, scan_body, jnp.int32(-1), unroll=8)

    pltpu.sync_copy(out_v, out_hbm.at[comp * _B + b])


def kernel(segment_values, segment_starts):
    vals_flat = segment_values.reshape(_B, _S * _C)
    out = _decode(vals_flat, segment_starts)
    return out.reshape(_C, _B, _T)
